# baseline (speedup 1.0000x reference)
import jax, jax.numpy as jnp
import numpy as np
from jax.experimental import pallas as pl

_SA1 = dict(npoint=512, radii=(0.1, 0.2, 0.4), nsamples=(16, 32, 128))
_SA2 = dict(npoint=128, radii=(0.2, 0.4, 0.8), nsamples=(32, 64, 128))


def _square_distance(src, dst):
    dist = -2.0 * jnp.matmul(src, jnp.transpose(dst, (0, 2, 1)))
    dist = dist + jnp.sum(src ** 2, -1)[:, :, None]
    dist = dist + jnp.sum(dst ** 2, -1)[:, None, :]
    return dist


def _index_points(points, idx):
    B = points.shape[0]
    batch = jnp.arange(B).reshape((B,) + (1,) * (idx.ndim - 1))
    return points[batch, idx]


def _fps(xyz, npoint):
    B, Np, _ = xyz.shape

    def body(i, state):
        centroids, distance, farthest = state
        centroids = centroids.at[:, i].set(farthest)
        centroid = jnp.take_along_axis(xyz, farthest[:, None, None], axis=1)
        dist = jnp.sum((xyz - centroid) ** 2, -1)
        distance = jnp.minimum(distance, dist)
        farthest = jnp.argmax(distance, axis=-1).astype(jnp.int32)
        return (centroids, distance, farthest)

    centroids = jnp.zeros((B, npoint), jnp.int32)
    distance = jnp.full((B, Np), 1e10, jnp.float32)
    farthest = jnp.zeros((B,), jnp.int32)
    centroids, _, _ = jax.lax.fori_loop(0, npoint, body, (centroids, distance, farthest))
    return centroids


def _query_ball(radius, nsample, xyz, new_xyz):
    B, Np, _ = xyz.shape
    S = new_xyz.shape[1]
    sqrdists = _square_distance(new_xyz, xyz)
    group_idx = jnp.broadcast_to(jnp.arange(Np, dtype=jnp.int32), (B, S, Np))
    group_idx = jnp.where(sqrdists > radius ** 2, Np, group_idx)
    group_idx = jnp.sort(group_idx, axis=-1)[:, :, :nsample]
    group_first = jnp.broadcast_to(group_idx[:, :, 0:1], group_idx.shape)
    group_idx = jnp.where(group_idx == Np, group_first, group_idx)
    return group_idx


def _apply_mlp(x, layers):
    for layer in layers:
        x = jnp.einsum('oc,bcks->boks', layer["W"], x) + layer["b"][None, :, None, None]
        x = x / jnp.sqrt(1.0 + 1e-5)
        x = x * layer["gamma"][None, :, None, None] + layer["beta"][None, :, None, None]
        x = jax.nn.relu(x)
    return x


def _sa_msg(xyz_cf, points_cf, npoint, radii, nsamples, branches):
    xyz = jnp.transpose(xyz_cf, (0, 2, 1))
    points = jnp.transpose(points_cf, (0, 2, 1)) if points_cf is not None else None
    fps_idx = _fps(jax.lax.stop_gradient(xyz), npoint)
    new_xyz = _index_points(xyz, fps_idx)
    outs = []
    for radius, K, layers in zip(radii, nsamples, branches):
        group_idx = _query_ball(radius, K, xyz, new_xyz)
        grouped_xyz = _index_points(xyz, group_idx) - new_xyz[:, :, None, :]
        if points is not None:
            grouped = jnp.concatenate([_index_points(points, group_idx), grouped_xyz], axis=-1)
        else:
            grouped = grouped_xyz
        g = jnp.transpose(grouped, (0, 3, 2, 1))
        g = _apply_mlp(g, layers)
        outs.append(jnp.max(g, axis=2))
    return jnp.transpose(new_xyz, (0, 2, 1)), jnp.concatenate(outs, axis=1)


def _copy_body(x_ref, o_ref):
    o_ref[...] = x_ref[...]


def kernel(xyz, params):
    Bc = xyz.shape[0]
    norm = xyz[:, 3:, :]
    xyz3 = xyz[:, :3, :]
    l1_xyz, l1_points = _sa_msg(xyz3, norm, _SA1["npoint"], _SA1["radii"], _SA1["nsamples"], params["sa1"])
    l2_xyz, l2_points = _sa_msg(l1_xyz, l1_points, _SA2["npoint"], _SA2["radii"], _SA2["nsamples"], params["sa2"])
    xyzT = jnp.transpose(l2_xyz, (0, 2, 1))
    ptsT = jnp.transpose(l2_points, (0, 2, 1))
    new_points = jnp.concatenate([xyzT, ptsT], axis=-1)[:, None, :, :]
    g = jnp.transpose(new_points, (0, 3, 2, 1))
    g = _apply_mlp(g, params["sa3"])
    l3 = jnp.max(g, axis=2).reshape(Bc, 128)
    return pl.pallas_call(
        _copy_body,
        out_shape=jax.ShapeDtypeStruct(l3.shape, l3.dtype),
    )(l3)


# speedup vs baseline: 23.4869x; 23.4869x over previous
import functools

import jax
import jax.numpy as jnp
import numpy as np
from jax import lax
from jax.experimental import pallas as pl
from jax.experimental.pallas import tpu as pltpu
from jax.experimental.pallas import tpu_sc as plsc

_SA1_RADII = (0.1, 0.2, 0.4)
_SA1_K = (16, 32, 128)
_SA2_RADII = (0.2, 0.4, 0.8)
_SA2_K = (32, 64, 128)

_NUM_WORKERS = 32


def _fps_body(pts_ref, newxyz_ref, *, npoint):
    B, _, Np = pts_ref.shape
    x = pts_ref[:, 0, :]
    y = pts_ref[:, 1, :]
    z = pts_ref[:, 2, :]
    iota = lax.broadcasted_iota(jnp.int32, (B, Np), 1)
    siota = lax.broadcasted_iota(jnp.int32, (1, 1, npoint), 2)

    def body(i, carry):
        distance, far, acc = carry
        sel = iota == far
        cx = jnp.sum(jnp.where(sel, x, 0.0), axis=1, keepdims=True)
        cy = jnp.sum(jnp.where(sel, y, 0.0), axis=1, keepdims=True)
        cz = jnp.sum(jnp.where(sel, z, 0.0), axis=1, keepdims=True)
        cxyz = jnp.concatenate([cx, cy, cz], axis=1)[:, :, None]
        acc = jnp.where(siota == i, cxyz, acc)
        d = (x - cx) ** 2 + (y - cy) ** 2
        d = d + (z - cz) ** 2
        distance = jnp.minimum(distance, d)
        m = jnp.max(distance, axis=1, keepdims=True)
        far = jnp.min(jnp.where(distance == m, iota, Np), axis=1, keepdims=True)
        return distance, far, acc

    dist0 = jnp.full((B, Np), 1e10, jnp.float32)
    far0 = jnp.zeros((B, 1), jnp.int32)
    acc0 = jnp.zeros((B, 3, npoint), jnp.float32)
    _, _, acc = lax.fori_loop(0, npoint, body, (dist0, far0, acc0))
    newxyz_ref[...] = acc


def _fps(pts_cf, npoint):
    B = pts_cf.shape[0]
    return pl.pallas_call(
        functools.partial(_fps_body, npoint=npoint),
        out_shape=jax.ShapeDtypeStruct((B, 3, npoint), jnp.float32),
    )(pts_cf)


def _sqd_body(cen_ref, pts_ref, d_ref):
    cen = cen_ref[0]
    ptsT = pts_ref[0]
    mm = jnp.dot(cen, ptsT, preferred_element_type=jnp.float32)
    c0 = cen[:, 0:1]
    c1 = cen[:, 1:2]
    c2 = cen[:, 2:3]
    s2 = c0 * c0 + c1 * c1
    s2 = s2 + c2 * c2
    p0 = ptsT[0:1, :]
    p1 = ptsT[1:2, :]
    p2 = ptsT[2:3, :]
    x2 = p0 * p0 + p1 * p1
    x2 = x2 + p2 * p2
    d = -2.0 * mm
    d = d + s2
    d = d + x2
    d_ref[0] = d


def _sqdist(cen, pts_cf):
    B, S, _ = cen.shape
    Np = pts_cf.shape[2]
    return pl.pallas_call(
        _sqd_body,
        grid=(B,),
        in_specs=[
            pl.BlockSpec((1, S, 3), lambda b: (b, 0, 0)),
            pl.BlockSpec((1, 3, Np), lambda b: (b, 0, 0)),
        ],
        out_specs=pl.BlockSpec((1, S, Np), lambda b: (b, 0, 0)),
        out_shape=jax.ShapeDtypeStruct((B, S, Np), jnp.float32),
    )(cen, pts_cf)


def _make_select(BS, S, Np, Ks, r2s):
    rpw = BS // _NUM_WORKERS
    kpads = tuple(k + 16 for k in Ks)
    mesh = plsc.VectorSubcoreMesh(core_axis_name="c", subcore_axis_name="s", num_cores=2, num_subcores=16)
    out_type = tuple(jax.ShapeDtypeStruct((BS, kp), jnp.int32) for kp in kpads)
    scratch = [pltpu.VMEM((Np,), jnp.float32)] + [
        pltpu.VMEM((rpw, kp), jnp.int32) for kp in kpads
    ]

    @functools.partial(
        pl.kernel, out_type=out_type, mesh=mesh, scratch_types=scratch,
        compiler_params=pltpu.CompilerParams(
            use_tc_tiling_on_sc=False, needs_layout_passes=False))
    def sel(d_hbm, o1, o2, o3, drow, buf1, buf2, buf3):
        bufs = (buf1, buf2, buf3)
        outs = (o1, o2, o3)
        wid = lax.axis_index("s") * 2 + lax.axis_index("c")
        row0 = wid * rpw
        bias = (row0 // S) * Np
        iota16 = lax.iota(jnp.int32, 16)

        def per_row(i, _):
            pltpu.sync_copy(d_hbm.at[row0 + i], drow)

            def chunk(c, cnts):
                dv = drow[pl.ds(c * 16, 16)]
                m3 = dv <= r2s[2]
                n3 = jnp.sum(m3.astype(jnp.int32))

                def heavy(cnts):
                    new = []
                    vals = c * 16 + iota16 + bias
                    for r in range(3):
                        m = dv <= r2s[r] if r < 2 else m3
                        mi = m.astype(jnp.int32)
                        pos = plsc.cumsum(mi) - mi
                        ok = jnp.logical_and(m, cnts[r] + pos < Ks[r])
                        plsc.store_compressed(
                            bufs[r].at[i].at[pl.ds(cnts[r], 16)], vals,
                            mask=ok)
                        new.append(cnts[r] + jnp.sum(ok.astype(jnp.int32)))
                    return tuple(new)

                return lax.cond(n3 > 0, heavy, lambda c_: c_, cnts)

            cnts = lax.fori_loop(
                0, Np // 16, chunk,
                (jnp.int32(0), jnp.int32(0), jnp.int32(0)))

            for r in range(3):
                fv = bufs[r][i, pl.ds(0, 16)][0]
                first = jnp.where(cnts[r] == 0, bias + Np - 1, fv)
                for cp in range(kpads[r] // 16):
                    lanes = cp * 16 + iota16
                    cur = bufs[r][i, pl.ds(cp * 16, 16)]
                    bufs[r][i, pl.ds(cp * 16, 16)] = jnp.where(
                        lanes < cnts[r], cur, first)
            return 0

        lax.fori_loop(0, rpw, per_row, 0)
        for r in range(3):
            pltpu.sync_copy(bufs[r], outs[r].at[pl.ds(row0, rpw)])

    return sel


_GCHUNK = 128


def _make_gather(N, D):
    per_w = N // _NUM_WORKERS
    nch = per_w // _GCHUNK
    assert per_w % _GCHUNK == 0
    mesh = plsc.VectorSubcoreMesh(core_axis_name="c", subcore_axis_name="s", num_cores=2, num_subcores=16)

    def body(table_hbm, idx_hbm, out_hbm, idx_v, rows_v, sem):
        wid = lax.axis_index("s") * 2 + lax.axis_index("c")
        base = wid * nch

        def step(j, _):
            row = base + j
            pltpu.sync_copy(idx_hbm.at[row], idx_v)
            pltpu.async_copy(table_hbm.at[idx_v], rows_v, sem).wait()
            pltpu.sync_copy(rows_v, out_hbm.at[pl.ds(row * _GCHUNK, _GCHUNK)])
            return 0

        lax.fori_loop(0, nch, step, 0)

    return functools.partial(
        pl.kernel, mesh=mesh,
        out_type=jax.ShapeDtypeStruct((N, D), jnp.float32),
        compiler_params=pltpu.CompilerParams(use_tc_tiling_on_sc=False),
        scratch_types=[
            pltpu.VMEM((_GCHUNK,), jnp.int32),
            pltpu.VMEM((_GCHUNK, D), jnp.float32),
            pltpu.SemaphoreType.DMA,
        ])(body)


def _gather(table, idx_flat):
    N = idx_flat.shape[0]
    D = table.shape[1]
    return _make_gather(N, D)(table, idx_flat.reshape(N // _GCHUNK, _GCHUNK))


def _mlp_body(g_ref, cen_ref, w1, b1, w2, b2, w3, b3, out_ref):
    ST, Kpad, D = g_ref.shape[1:]
    g = g_ref[0]
    cen = cen_ref[0]
    x = (g - cen[:, None, :]).reshape(ST * Kpad, D)
    x = jnp.maximum(jnp.dot(x, w1[...], preferred_element_type=jnp.float32)
                    + b1[...], 0.0)
    x = jnp.maximum(jnp.dot(x, w2[...], preferred_element_type=jnp.float32)
                    + b2[...], 0.0)
    x = jnp.maximum(jnp.dot(x, w3[...], preferred_element_type=jnp.float32)
                    + b3[...], 0.0)
    C3 = x.shape[1]
    out_ref[0] = jnp.max(x.reshape(ST, Kpad, C3), axis=1)


def _grouped_mlp(g, cen_pad, ws, st):
    B, S, Kpad, D = g.shape
    C3 = ws[2][0].shape[1]
    wspecs = []
    wargs = []
    for wT, b in ws:
        wspecs.append(pl.BlockSpec(wT.shape, lambda b_, s_: (0, 0)))
        wspecs.append(pl.BlockSpec(b.shape, lambda b_, s_: (0, 0)))
        wargs.extend([wT, b])
    return pl.pallas_call(
        _mlp_body,
        grid=(B, S // st),
        in_specs=[
            pl.BlockSpec((1, st, Kpad, D), lambda b, s: (b, s, 0, 0)),
            pl.BlockSpec((1, st, D), lambda b, s: (b, s, 0)),
        ] + wspecs,
        out_specs=pl.BlockSpec((1, st, C3), lambda b, s: (b, s, 0)),
        out_shape=jax.ShapeDtypeStruct((B, S, C3), jnp.float32),
    )(g, cen_pad, *wargs)


def _sa3_body(t_ref, w1, b1, w2, b2, w3, b3, out_ref):
    x = t_ref[0]
    x = jnp.maximum(jnp.dot(x, w1[...], preferred_element_type=jnp.float32)
                    + b1[...], 0.0)
    x = jnp.maximum(jnp.dot(x, w2[...], preferred_element_type=jnp.float32)
                    + b2[...], 0.0)
    x = jnp.maximum(jnp.dot(x, w3[...], preferred_element_type=jnp.float32)
                    + b3[...], 0.0)
    out_ref[0, 0] = jnp.max(x, axis=0)


def _sa3(t, ws):
    B, S, C = t.shape
    C3 = ws[2][0].shape[1]
    wspecs = []
    wargs = []
    for wT, b in ws:
        wspecs.append(pl.BlockSpec(wT.shape, lambda b_: (0, 0)))
        wspecs.append(pl.BlockSpec(b.shape, lambda b_: (0, 0)))
        wargs.extend([wT, b])
    return pl.pallas_call(
        _sa3_body,
        grid=(B,),
        in_specs=[pl.BlockSpec((1, S, C), lambda b: (b, 0, 0))] + wspecs,
        out_specs=pl.BlockSpec((1, 1, C3), lambda b: (b, 0, 0)),
        out_shape=jax.ShapeDtypeStruct((B, 1, C3), jnp.float32),
    )(t, *wargs).reshape(B, C3)


def _fold(layers):
    out = []
    for layer in layers:
        s = layer["gamma"] / jnp.sqrt(1.0 + 1e-5)
        wT = (layer["W"] * s[:, None]).T
        bf = (layer["b"] * s + layer["beta"])[None, :]
        out.append((wT, bf))
    return out


def _r2s(radii):
    return tuple(float(np.float32(np.float64(r) ** 2)) for r in radii)


def _sa_stage(pts_cf, table, dreal, npoint, radii, ks, branches, sts):
    B, _, Np = pts_cf.shape
    dpad = table.shape[1]
    nx_cf = _fps(pts_cf, npoint)
    nx = jnp.transpose(nx_cf, (0, 2, 1))
    d = _sqdist(nx, pts_cf)
    sel = _make_select(B * npoint, npoint, Np, ks, _r2s(radii))
    idxs = sel(d.reshape(B * npoint, Np))
    cen_pad = jnp.concatenate(
        [jnp.zeros((B, npoint, dreal - 3), jnp.float32), nx,
         jnp.zeros((B, npoint, dpad - dreal), jnp.float32)], axis=-1)
    outs = []
    for r in range(3):
        kpad = ks[r] + 16
        ws = branches[r]
        w1T, b1 = ws[0]
        w1T = jnp.concatenate(
            [w1T, jnp.zeros((dpad - dreal, w1T.shape[1]), jnp.float32)], 0)
        ws = [(w1T, b1), ws[1], ws[2]]
        g = _gather(table, idxs[r].reshape(-1))
        g = g.reshape(B, npoint, kpad, dpad)
        outs.append(_grouped_mlp(g, cen_pad, ws, sts[r]))
    return nx_cf, nx, jnp.concatenate(outs, axis=-1)


def kernel(xyz, params):
    B = xyz.shape[0]
    Np = xyz.shape[2]
    pts_cf = xyz[:, :3, :]
    ptsT = jnp.transpose(xyz, (0, 2, 1))
    table1 = jnp.concatenate(
        [ptsT[:, :, 3:], ptsT[:, :, :3],
         jnp.zeros((B, Np, 10), jnp.float32)], axis=-1).reshape(B * Np, 16)

    sa1 = [_fold(br) for br in params["sa1"]]
    sa2 = [_fold(br) for br in params["sa2"]]
    sa3 = _fold(params["sa3"])

    nx1_cf, nx1, l1p = _sa_stage(
        pts_cf, table1, 6, 512, _SA1_RADII, _SA1_K, sa1, (64, 64, 32))
    table2 = jnp.concatenate(
        [l1p, nx1, jnp.zeros((B, 512, 13), jnp.float32)],
        axis=-1).reshape(B * 512, 176)
    _, nx2, l2p = _sa_stage(
        nx1_cf, table2, 163, 128, _SA2_RADII, _SA2_K, sa2, (32, 32, 16))
    t3 = jnp.concatenate([nx2, l2p], axis=-1)
    return _sa3(t3, sa3)


# speedup vs baseline: 24.0697x; 1.0248x over previous
import functools

import jax
import jax.numpy as jnp
import numpy as np
from jax import lax
from jax.experimental import pallas as pl
from jax.experimental.pallas import tpu as pltpu
from jax.experimental.pallas import tpu_sc as plsc

_SA1_RADII = (0.1, 0.2, 0.4)
_SA1_K = (16, 32, 128)
_SA2_RADII = (0.2, 0.4, 0.8)
_SA2_K = (32, 64, 128)

_NUM_WORKERS = 32


def _fps_body(pts_ref, newxyz_ref, *, npoint):
    B, _, Np = pts_ref.shape
    x = pts_ref[:, 0, :]
    y = pts_ref[:, 1, :]
    z = pts_ref[:, 2, :]
    iota = lax.broadcasted_iota(jnp.int32, (B, Np), 1)
    siota = lax.broadcasted_iota(jnp.int32, (1, 1, npoint), 2)

    def body(i, carry):
        distance, far, acc = carry
        sel = iota == far
        cx = jnp.sum(jnp.where(sel, x, 0.0), axis=1, keepdims=True)
        cy = jnp.sum(jnp.where(sel, y, 0.0), axis=1, keepdims=True)
        cz = jnp.sum(jnp.where(sel, z, 0.0), axis=1, keepdims=True)
        cxyz = jnp.concatenate([cx, cy, cz], axis=1)[:, :, None]
        acc = jnp.where(siota == i, cxyz, acc)
        d = (x - cx) ** 2 + (y - cy) ** 2
        d = d + (z - cz) ** 2
        distance = jnp.minimum(distance, d)
        m = jnp.max(distance, axis=1, keepdims=True)
        far = jnp.min(jnp.where(distance == m, iota, Np), axis=1, keepdims=True)
        return distance, far, acc

    dist0 = jnp.full((B, Np), 1e10, jnp.float32)
    far0 = jnp.zeros((B, 1), jnp.int32)
    acc0 = jnp.zeros((B, 3, npoint), jnp.float32)
    _, _, acc = lax.fori_loop(0, npoint, body, (dist0, far0, acc0))
    newxyz_ref[...] = acc


def _fps(pts_cf, npoint):
    B = pts_cf.shape[0]
    return pl.pallas_call(
        functools.partial(_fps_body, npoint=npoint),
        out_shape=jax.ShapeDtypeStruct((B, 3, npoint), jnp.float32),
    )(pts_cf)


def _sqd_body(cen_ref, pts_ref, d_ref):
    cen = cen_ref[0]
    ptsT = pts_ref[0]
    mm = jnp.dot(cen, ptsT, preferred_element_type=jnp.float32)
    c0 = cen[:, 0:1]
    c1 = cen[:, 1:2]
    c2 = cen[:, 2:3]
    s2 = c0 * c0 + c1 * c1
    s2 = s2 + c2 * c2
    p0 = ptsT[0:1, :]
    p1 = ptsT[1:2, :]
    p2 = ptsT[2:3, :]
    x2 = p0 * p0 + p1 * p1
    x2 = x2 + p2 * p2
    d = -2.0 * mm
    d = d + s2
    d = d + x2
    d_ref[0] = d


def _sqdist(cen, pts_cf):
    B, S, _ = cen.shape
    Np = pts_cf.shape[2]
    return pl.pallas_call(
        _sqd_body,
        grid=(B,),
        in_specs=[
            pl.BlockSpec((1, S, 3), lambda b: (b, 0, 0)),
            pl.BlockSpec((1, 3, Np), lambda b: (b, 0, 0)),
        ],
        out_specs=pl.BlockSpec((1, S, Np), lambda b: (b, 0, 0)),
        out_shape=jax.ShapeDtypeStruct((B, S, Np), jnp.float32),
    )(cen, pts_cf)


def _make_select(BS, S, Np, Ks, r2s):
    rpw = BS // _NUM_WORKERS
    kpads = tuple(k + 16 for k in Ks)
    mesh = plsc.VectorSubcoreMesh(core_axis_name="c", subcore_axis_name="s", num_cores=2, num_subcores=16)
    out_type = tuple(jax.ShapeDtypeStruct((BS, kp), jnp.int32) for kp in kpads)
    scratch = [pltpu.VMEM((2, Np), jnp.float32)] + [
        pltpu.VMEM((rpw, kp), jnp.int32) for kp in kpads
    ] + [pltpu.SemaphoreType.DMA]

    @functools.partial(
        pl.kernel, out_type=out_type, mesh=mesh, scratch_types=scratch,
        compiler_params=pltpu.CompilerParams(
            use_tc_tiling_on_sc=False, needs_layout_passes=False))
    def sel(d_hbm, o1, o2, o3, drow, buf1, buf2, buf3, sem):
        bufs = (buf1, buf2, buf3)
        outs = (o1, o2, o3)
        wid = lax.axis_index("s") * 2 + lax.axis_index("c")
        row0 = wid * rpw
        bias = (row0 // S) * Np
        iota16 = lax.iota(jnp.int32, 16)

        pltpu.async_copy(d_hbm.at[row0], drow.at[0], sem)

        def per_row(i, _):
            cur = lax.rem(i, 2)
            pltpu.make_async_copy(d_hbm.at[row0 + i], drow.at[cur], sem).wait()

            @pl.when(i + 1 < rpw)
            def _prefetch():
                pltpu.async_copy(
                    d_hbm.at[row0 + i + 1], drow.at[1 - cur], sem)

            def chunk(c, cnts):
                dv = drow.at[cur][pl.ds(c * 16, 16)]
                m3 = dv <= r2s[2]
                n3 = plsc.all_reduce_population_count(m3)[0]

                def heavy(cnts):
                    new = []
                    vals = c * 16 + iota16 + bias
                    for r in range(3):
                        m = dv <= r2s[r] if r < 2 else m3
                        nm = (n3 if r == 2
                              else plsc.all_reduce_population_count(m)[0])
                        cnt = cnts[r]

                        def fast(cnt, m=m, r=r, nm=nm):
                            plsc.store_compressed(
                                bufs[r].at[i].at[pl.ds(cnt, 16)], vals,
                                mask=m)
                            return cnt + nm

                        def slow(cnt, m=m, r=r):
                            mi = m.astype(jnp.int32)
                            pos = plsc.cumsum(mi) - mi
                            ok = jnp.logical_and(m, cnt + pos < Ks[r])
                            plsc.store_compressed(
                                bufs[r].at[i].at[pl.ds(cnt, 16)], vals,
                                mask=ok)
                            return jnp.int32(Ks[r])

                        new.append(
                            lax.cond(cnt + nm <= Ks[r], fast, slow, cnt))
                    return tuple(new)

                return lax.cond(n3 > 0, heavy, lambda c_: c_, cnts)

            cnts = lax.fori_loop(
                0, Np // 16, chunk,
                (jnp.int32(0), jnp.int32(0), jnp.int32(0)))

            for r in range(3):
                fv = bufs[r][i, pl.ds(0, 16)][0]
                first = jnp.where(cnts[r] == 0, bias + Np - 1, fv)
                for cp in range(kpads[r] // 16):
                    lanes = cp * 16 + iota16
                    cur = bufs[r][i, pl.ds(cp * 16, 16)]
                    bufs[r][i, pl.ds(cp * 16, 16)] = jnp.where(
                        lanes < cnts[r], cur, first)
            return 0

        lax.fori_loop(0, rpw, per_row, 0)
        for r in range(3):
            pltpu.sync_copy(bufs[r], outs[r].at[pl.ds(row0, rpw)])

    return sel


_GCHUNK = 128


def _make_gather(N, D):
    per_w = N // _NUM_WORKERS
    nch = per_w // _GCHUNK
    assert per_w % _GCHUNK == 0
    mesh = plsc.VectorSubcoreMesh(core_axis_name="c", subcore_axis_name="s", num_cores=2, num_subcores=16)

    grp = 4
    while nch % grp:
        grp //= 2
    ngr = nch // grp

    def body(table_hbm, idx_hbm, out_hbm, idx_v, rows_v, sem):
        wid = lax.axis_index("s") * 2 + lax.axis_index("c")
        base = wid * nch

        def step(g, _):
            row = base + g * grp
            pltpu.sync_copy(idx_hbm.at[pl.ds(row, grp)], idx_v)
            for k in range(grp):
                pltpu.async_copy(table_hbm.at[idx_v.at[k]], rows_v.at[k], sem)
            for k in range(grp):
                pltpu.make_async_copy(
                    table_hbm.at[idx_v.at[k]], rows_v.at[k], sem).wait()
            for k in range(grp):
                pltpu.async_copy(
                    rows_v.at[k],
                    out_hbm.at[pl.ds((row + k) * _GCHUNK, _GCHUNK)], sem)
            for k in range(grp):
                pltpu.make_async_copy(
                    rows_v.at[k],
                    out_hbm.at[pl.ds((row + k) * _GCHUNK, _GCHUNK)],
                    sem).wait()
            return 0

        lax.fori_loop(0, ngr, step, 0)

    return functools.partial(
        pl.kernel, mesh=mesh,
        out_type=jax.ShapeDtypeStruct((N, D), jnp.float32),
        compiler_params=pltpu.CompilerParams(use_tc_tiling_on_sc=False),
        scratch_types=[
            pltpu.VMEM((grp, _GCHUNK), jnp.int32),
            pltpu.VMEM((grp, _GCHUNK, D), jnp.float32),
            pltpu.SemaphoreType.DMA,
        ])(body)


def _gather(table, idx_flat):
    N = idx_flat.shape[0]
    D = table.shape[1]
    return _make_gather(N, D)(table, idx_flat.reshape(N // _GCHUNK, _GCHUNK))


def _mlp_body(g_ref, cen_ref, w1, b1, w2, b2, w3, b3, out_ref):
    ST, Kpad, D = g_ref.shape[1:]
    g = g_ref[0]
    cen = cen_ref[0]
    x = (g - cen[:, None, :]).reshape(ST * Kpad, D)
    x = jnp.maximum(jnp.dot(x, w1[...], preferred_element_type=jnp.float32)
                    + b1[...], 0.0)
    x = jnp.maximum(jnp.dot(x, w2[...], preferred_element_type=jnp.float32)
                    + b2[...], 0.0)
    x = jnp.maximum(jnp.dot(x, w3[...], preferred_element_type=jnp.float32)
                    + b3[...], 0.0)
    C3 = x.shape[1]
    out_ref[0] = jnp.max(x.reshape(ST, Kpad, C3), axis=1)


def _grouped_mlp(g, cen_pad, ws, st):
    B, S, Kpad, D = g.shape
    C3 = ws[2][0].shape[1]
    wspecs = []
    wargs = []
    for wT, b in ws:
        wspecs.append(pl.BlockSpec(wT.shape, lambda b_, s_: (0, 0)))
        wspecs.append(pl.BlockSpec(b.shape, lambda b_, s_: (0, 0)))
        wargs.extend([wT, b])
    return pl.pallas_call(
        _mlp_body,
        grid=(B, S // st),
        in_specs=[
            pl.BlockSpec((1, st, Kpad, D), lambda b, s: (b, s, 0, 0)),
            pl.BlockSpec((1, st, D), lambda b, s: (b, s, 0)),
        ] + wspecs,
        out_specs=pl.BlockSpec((1, st, C3), lambda b, s: (b, s, 0)),
        out_shape=jax.ShapeDtypeStruct((B, S, C3), jnp.float32),
    )(g, cen_pad, *wargs)


def _sa3_body(t_ref, w1, b1, w2, b2, w3, b3, out_ref):
    x = t_ref[0]
    x = jnp.maximum(jnp.dot(x, w1[...], preferred_element_type=jnp.float32)
                    + b1[...], 0.0)
    x = jnp.maximum(jnp.dot(x, w2[...], preferred_element_type=jnp.float32)
                    + b2[...], 0.0)
    x = jnp.maximum(jnp.dot(x, w3[...], preferred_element_type=jnp.float32)
                    + b3[...], 0.0)
    out_ref[0, 0] = jnp.max(x, axis=0)


def _sa3(t, ws):
    B, S, C = t.shape
    C3 = ws[2][0].shape[1]
    wspecs = []
    wargs = []
    for wT, b in ws:
        wspecs.append(pl.BlockSpec(wT.shape, lambda b_: (0, 0)))
        wspecs.append(pl.BlockSpec(b.shape, lambda b_: (0, 0)))
        wargs.extend([wT, b])
    return pl.pallas_call(
        _sa3_body,
        grid=(B,),
        in_specs=[pl.BlockSpec((1, S, C), lambda b: (b, 0, 0))] + wspecs,
        out_specs=pl.BlockSpec((1, 1, C3), lambda b: (b, 0, 0)),
        out_shape=jax.ShapeDtypeStruct((B, 1, C3), jnp.float32),
    )(t, *wargs).reshape(B, C3)


def _fold(layers):
    out = []
    for layer in layers:
        s = layer["gamma"] / jnp.sqrt(1.0 + 1e-5)
        wT = (layer["W"] * s[:, None]).T
        bf = (layer["b"] * s + layer["beta"])[None, :]
        out.append((wT, bf))
    return out


def _r2s(radii):
    return tuple(float(np.float32(np.float64(r) ** 2)) for r in radii)


def _sa_stage(pts_cf, table, dreal, npoint, radii, ks, branches, sts):
    B, _, Np = pts_cf.shape
    dpad = table.shape[1]
    nx_cf = _fps(pts_cf, npoint)
    nx = jnp.transpose(nx_cf, (0, 2, 1))
    d = _sqdist(nx, pts_cf)
    sel = _make_select(B * npoint, npoint, Np, ks, _r2s(radii))
    idxs = sel(d.reshape(B * npoint, Np))
    cen_pad = jnp.concatenate(
        [jnp.zeros((B, npoint, dreal - 3), jnp.float32), nx,
         jnp.zeros((B, npoint, dpad - dreal), jnp.float32)], axis=-1)
    outs = []
    for r in range(3):
        kpad = ks[r] + 16
        ws = branches[r]
        w1T, b1 = ws[0]
        w1T = jnp.concatenate(
            [w1T, jnp.zeros((dpad - dreal, w1T.shape[1]), jnp.float32)], 0)
        ws = [(w1T, b1), ws[1], ws[2]]
        g = _gather(table, idxs[r].reshape(-1))
        g = g.reshape(B, npoint, kpad, dpad)
        outs.append(_grouped_mlp(g, cen_pad, ws, sts[r]))
    return nx_cf, nx, jnp.concatenate(outs, axis=-1)


def kernel(xyz, params):
    B = xyz.shape[0]
    Np = xyz.shape[2]
    pts_cf = xyz[:, :3, :]
    ptsT = jnp.transpose(xyz, (0, 2, 1))
    table1 = jnp.concatenate(
        [ptsT[:, :, 3:], ptsT[:, :, :3],
         jnp.zeros((B, Np, 10), jnp.float32)], axis=-1).reshape(B * Np, 16)

    sa1 = [_fold(br) for br in params["sa1"]]
    sa2 = [_fold(br) for br in params["sa2"]]
    sa3 = _fold(params["sa3"])

    nx1_cf, nx1, l1p = _sa_stage(
        pts_cf, table1, 6, 512, _SA1_RADII, _SA1_K, sa1, (64, 64, 32))
    table2 = jnp.concatenate(
        [l1p, nx1, jnp.zeros((B, 512, 13), jnp.float32)],
        axis=-1).reshape(B * 512, 176)
    _, nx2, l2p = _sa_stage(
        nx1_cf, table2, 163, 128, _SA2_RADII, _SA2_K, sa2, (32, 32, 16))
    t3 = jnp.concatenate([nx2, l2p], axis=-1)
    return _sa3(t3, sa3)


# speedup vs baseline: 33.2990x; 1.3834x over previous
import functools

import jax
import jax.numpy as jnp
import numpy as np
from jax import lax
from jax.experimental import pallas as pl
from jax.experimental.pallas import tpu as pltpu
from jax.experimental.pallas import tpu_sc as plsc

_SA1_RADII = (0.1, 0.2, 0.4)
_SA1_K = (16, 32, 128)
_SA2_RADII = (0.2, 0.4, 0.8)
_SA2_K = (32, 64, 128)

_NUM_WORKERS = 32


def _fps_body(pts_ref, newxyz_ref, *, npoint):
    B, _, Np = pts_ref.shape
    x = pts_ref[:, 0, :]
    y = pts_ref[:, 1, :]
    z = pts_ref[:, 2, :]
    iota = lax.broadcasted_iota(jnp.int32, (B, Np), 1)
    siota = lax.broadcasted_iota(jnp.int32, (1, 1, npoint), 2)

    def body(i, carry):
        distance, far, acc = carry
        sel = iota == far
        cx = jnp.sum(jnp.where(sel, x, 0.0), axis=1, keepdims=True)
        cy = jnp.sum(jnp.where(sel, y, 0.0), axis=1, keepdims=True)
        cz = jnp.sum(jnp.where(sel, z, 0.0), axis=1, keepdims=True)
        cxyz = jnp.concatenate([cx, cy, cz], axis=1)[:, :, None]
        acc = jnp.where(siota == i, cxyz, acc)
        d = (x - cx) ** 2 + (y - cy) ** 2
        d = d + (z - cz) ** 2
        distance = jnp.minimum(distance, d)
        m = jnp.max(distance, axis=1, keepdims=True)
        far = jnp.min(jnp.where(distance == m, iota, Np), axis=1, keepdims=True)
        return distance, far, acc

    dist0 = jnp.full((B, Np), 1e10, jnp.float32)
    far0 = jnp.zeros((B, 1), jnp.int32)
    acc0 = jnp.zeros((B, 3, npoint), jnp.float32)
    _, _, acc = lax.fori_loop(0, npoint, body, (dist0, far0, acc0))
    newxyz_ref[...] = acc


def _fps(pts_cf, npoint):
    B = pts_cf.shape[0]
    return pl.pallas_call(
        functools.partial(_fps_body, npoint=npoint),
        out_shape=jax.ShapeDtypeStruct((B, 3, npoint), jnp.float32),
    )(pts_cf)


def _sqd_body(cen_ref, pts_ref, d_ref):
    cen = cen_ref[0]
    ptsT = pts_ref[0]
    mm = jnp.dot(cen, ptsT, preferred_element_type=jnp.float32)
    c0 = cen[:, 0:1]
    c1 = cen[:, 1:2]
    c2 = cen[:, 2:3]
    s2 = c0 * c0 + c1 * c1
    s2 = s2 + c2 * c2
    p0 = ptsT[0:1, :]
    p1 = ptsT[1:2, :]
    p2 = ptsT[2:3, :]
    x2 = p0 * p0 + p1 * p1
    x2 = x2 + p2 * p2
    d = -2.0 * mm
    d = d + s2
    d = d + x2
    d_ref[0] = d


def _sqdist(cen, pts_cf):
    B, S, _ = cen.shape
    Np = pts_cf.shape[2]
    return pl.pallas_call(
        _sqd_body,
        grid=(B,),
        in_specs=[
            pl.BlockSpec((1, S, 3), lambda b: (b, 0, 0)),
            pl.BlockSpec((1, 3, Np), lambda b: (b, 0, 0)),
        ],
        out_specs=pl.BlockSpec((1, S, Np), lambda b: (b, 0, 0)),
        out_shape=jax.ShapeDtypeStruct((B, S, Np), jnp.float32),
    )(cen, pts_cf)


def _make_select(BS, S, Np, Ks, r2s):
    rpw = BS // _NUM_WORKERS
    kpads = tuple(k + 16 for k in Ks)
    mesh = plsc.VectorSubcoreMesh(core_axis_name="c", subcore_axis_name="s", num_cores=2, num_subcores=16)
    out_type = tuple(jax.ShapeDtypeStruct((BS, kp), jnp.int32) for kp in kpads)
    scratch = [pltpu.VMEM((2, Np), jnp.float32)] + [
        pltpu.VMEM((rpw, kp), jnp.int32) for kp in kpads
    ] + [pltpu.VMEM((Np + 16,), jnp.int32),
         pltpu.VMEM((Np + 16,), jnp.float32),
         pltpu.SemaphoreType.DMA]

    @functools.partial(
        pl.kernel, out_type=out_type, mesh=mesh, scratch_types=scratch,
        compiler_params=pltpu.CompilerParams(
            use_tc_tiling_on_sc=False, needs_layout_passes=False))
    def sel(d_hbm, o1, o2, o3, drow, buf1, buf2, buf3, cand, dcand, sem):
        bufs = (buf1, buf2, buf3)
        outs = (o1, o2, o3)
        wid = lax.axis_index("s") * 2 + lax.axis_index("c")
        row0 = wid * rpw
        bias = (row0 // S) * Np
        iota16 = lax.iota(jnp.int32, 16)

        pltpu.async_copy(d_hbm.at[row0], drow.at[0], sem)

        def per_row(i, _):
            cur = lax.rem(i, 2)
            pltpu.make_async_copy(d_hbm.at[row0 + i], drow.at[cur], sem).wait()

            @pl.when(i + 1 < rpw)
            def _prefetch():
                pltpu.async_copy(
                    d_hbm.at[row0 + i + 1], drow.at[1 - cur], sem)

            def p1(c4, nc):
                for u in range(4):
                    c = c4 * 4 + u
                    dv = drow.at[cur][pl.ds(c * 16, 16)]
                    m3 = dv <= r2s[2]
                    plsc.store_compressed(
                        cand.at[pl.ds(nc, 16)], c * 16 + iota16, mask=m3)
                    plsc.store_compressed(
                        dcand.at[pl.ds(nc, 16)], dv, mask=m3)
                    nc = nc + plsc.all_reduce_population_count(m3)[0]
                return nc

            nc = lax.fori_loop(0, Np // 64, p1, jnp.int32(0))

            def p2(k, cnts):
                lanes = k * 16 + iota16
                dv = dcand[pl.ds(k * 16, 16)]
                vals = cand[pl.ds(k * 16, 16)] + bias
                valid = lanes < nc
                new = []
                for r in range(3):
                    m = jnp.logical_and(dv <= r2s[r], valid)
                    mi = m.astype(jnp.int32)
                    pos = plsc.cumsum(mi) - mi
                    ok = jnp.logical_and(m, cnts[r] + pos < Ks[r])
                    plsc.store_compressed(
                        bufs[r].at[i].at[pl.ds(cnts[r], 16)], vals, mask=ok)
                    new.append(
                        cnts[r] + plsc.all_reduce_population_count(ok)[0])
                return tuple(new)

            cnts = lax.fori_loop(
                0, (nc + 15) // 16, p2,
                (jnp.int32(0), jnp.int32(0), jnp.int32(0)))

            for r in range(3):
                fv = bufs[r][i, pl.ds(0, 16)][0]
                first = jnp.where(cnts[r] == 0, bias + Np - 1, fv)
                for cp in range(kpads[r] // 16):
                    lanes = cp * 16 + iota16
                    cur = bufs[r][i, pl.ds(cp * 16, 16)]
                    bufs[r][i, pl.ds(cp * 16, 16)] = jnp.where(
                        lanes < cnts[r], cur, first)
            return 0

        lax.fori_loop(0, rpw, per_row, 0)
        for r in range(3):
            pltpu.sync_copy(bufs[r], outs[r].at[pl.ds(row0, rpw)])

    return sel


_GCHUNK = 128


def _make_gather(N, D):
    per_w = N // _NUM_WORKERS
    nch = per_w // _GCHUNK
    assert per_w % _GCHUNK == 0
    mesh = plsc.VectorSubcoreMesh(core_axis_name="c", subcore_axis_name="s", num_cores=2, num_subcores=16)

    grp = 4
    while nch % grp:
        grp //= 2
    ngr = nch // grp

    def body(table_hbm, idx_hbm, out_hbm, idx_v, rows_v, sem):
        wid = lax.axis_index("s") * 2 + lax.axis_index("c")
        base = wid * nch

        def step(g, _):
            row = base + g * grp
            pltpu.sync_copy(idx_hbm.at[pl.ds(row, grp)], idx_v)
            for k in range(grp):
                pltpu.async_copy(table_hbm.at[idx_v.at[k]], rows_v.at[k], sem)
            for k in range(grp):
                pltpu.make_async_copy(
                    table_hbm.at[idx_v.at[k]], rows_v.at[k], sem).wait()
            for k in range(grp):
                pltpu.async_copy(
                    rows_v.at[k],
                    out_hbm.at[pl.ds((row + k) * _GCHUNK, _GCHUNK)], sem)
            for k in range(grp):
                pltpu.make_async_copy(
                    rows_v.at[k],
                    out_hbm.at[pl.ds((row + k) * _GCHUNK, _GCHUNK)],
                    sem).wait()
            return 0

        lax.fori_loop(0, ngr, step, 0)

    return functools.partial(
        pl.kernel, mesh=mesh,
        out_type=jax.ShapeDtypeStruct((N, D), jnp.float32),
        compiler_params=pltpu.CompilerParams(use_tc_tiling_on_sc=False),
        scratch_types=[
            pltpu.VMEM((grp, _GCHUNK), jnp.int32),
            pltpu.VMEM((grp, _GCHUNK, D), jnp.float32),
            pltpu.SemaphoreType.DMA,
        ])(body)


def _gather(table, idx_flat):
    N = idx_flat.shape[0]
    D = table.shape[1]
    return _make_gather(N, D)(table, idx_flat.reshape(N // _GCHUNK, _GCHUNK))


def _mlp_body(g_ref, cen_ref, w1, b1, w2, b2, w3, b3, out_ref):
    ST, Kpad, D = g_ref.shape[1:]
    g = g_ref[0]
    cen = cen_ref[0]
    x = (g - cen[:, None, :]).reshape(ST * Kpad, D)
    x = jnp.maximum(jnp.dot(x, w1[...], preferred_element_type=jnp.float32)
                    + b1[...], 0.0)
    x = jnp.maximum(jnp.dot(x, w2[...], preferred_element_type=jnp.float32)
                    + b2[...], 0.0)
    x = jnp.maximum(jnp.dot(x, w3[...], preferred_element_type=jnp.float32)
                    + b3[...], 0.0)
    C3 = x.shape[1]
    out_ref[0] = jnp.max(x.reshape(ST, Kpad, C3), axis=1)


def _grouped_mlp(g, cen_pad, ws, st):
    B, S, Kpad, D = g.shape
    C3 = ws[2][0].shape[1]
    wspecs = []
    wargs = []
    for wT, b in ws:
        wspecs.append(pl.BlockSpec(wT.shape, lambda b_, s_: (0, 0)))
        wspecs.append(pl.BlockSpec(b.shape, lambda b_, s_: (0, 0)))
        wargs.extend([wT, b])
    return pl.pallas_call(
        _mlp_body,
        grid=(B, S // st),
        in_specs=[
            pl.BlockSpec((1, st, Kpad, D), lambda b, s: (b, s, 0, 0)),
            pl.BlockSpec((1, st, D), lambda b, s: (b, s, 0)),
        ] + wspecs,
        out_specs=pl.BlockSpec((1, st, C3), lambda b, s: (b, s, 0)),
        out_shape=jax.ShapeDtypeStruct((B, S, C3), jnp.float32),
    )(g, cen_pad, *wargs)


def _sa3_body(t_ref, w1, b1, w2, b2, w3, b3, out_ref):
    x = t_ref[0]
    x = jnp.maximum(jnp.dot(x, w1[...], preferred_element_type=jnp.float32)
                    + b1[...], 0.0)
    x = jnp.maximum(jnp.dot(x, w2[...], preferred_element_type=jnp.float32)
                    + b2[...], 0.0)
    x = jnp.maximum(jnp.dot(x, w3[...], preferred_element_type=jnp.float32)
                    + b3[...], 0.0)
    out_ref[0, 0] = jnp.max(x, axis=0)


def _sa3(t, ws):
    B, S, C = t.shape
    C3 = ws[2][0].shape[1]
    wspecs = []
    wargs = []
    for wT, b in ws:
        wspecs.append(pl.BlockSpec(wT.shape, lambda b_: (0, 0)))
        wspecs.append(pl.BlockSpec(b.shape, lambda b_: (0, 0)))
        wargs.extend([wT, b])
    return pl.pallas_call(
        _sa3_body,
        grid=(B,),
        in_specs=[pl.BlockSpec((1, S, C), lambda b: (b, 0, 0))] + wspecs,
        out_specs=pl.BlockSpec((1, 1, C3), lambda b: (b, 0, 0)),
        out_shape=jax.ShapeDtypeStruct((B, 1, C3), jnp.float32),
    )(t, *wargs).reshape(B, C3)


def _fold(layers):
    out = []
    for layer in layers:
        s = layer["gamma"] / jnp.sqrt(1.0 + 1e-5)
        wT = (layer["W"] * s[:, None]).T
        bf = (layer["b"] * s + layer["beta"])[None, :]
        out.append((wT, bf))
    return out


def _r2s(radii):
    return tuple(float(np.float32(np.float64(r) ** 2)) for r in radii)


def _sa_stage(pts_cf, table, dreal, npoint, radii, ks, branches, sts):
    B, _, Np = pts_cf.shape
    dpad = table.shape[1]
    nx_cf = _fps(pts_cf, npoint)
    nx = jnp.transpose(nx_cf, (0, 2, 1))
    d = _sqdist(nx, pts_cf)
    sel = _make_select(B * npoint, npoint, Np, ks, _r2s(radii))
    idxs = sel(d.reshape(B * npoint, Np))
    cen_pad = jnp.concatenate(
        [jnp.zeros((B, npoint, dreal - 3), jnp.float32), nx,
         jnp.zeros((B, npoint, dpad - dreal), jnp.float32)], axis=-1)
    outs = []
    for r in range(3):
        kpad = ks[r] + 16
        ws = branches[r]
        w1T, b1 = ws[0]
        w1T = jnp.concatenate(
            [w1T, jnp.zeros((dpad - dreal, w1T.shape[1]), jnp.float32)], 0)
        ws = [(w1T, b1), ws[1], ws[2]]
        g = _gather(table, idxs[r].reshape(-1))
        g = g.reshape(B, npoint, kpad, dpad)
        outs.append(_grouped_mlp(g, cen_pad, ws, sts[r]))
    return nx_cf, nx, jnp.concatenate(outs, axis=-1)


def kernel(xyz, params):
    B = xyz.shape[0]
    Np = xyz.shape[2]
    pts_cf = xyz[:, :3, :]
    ptsT = jnp.transpose(xyz, (0, 2, 1))
    table1 = jnp.concatenate(
        [ptsT[:, :, 3:], ptsT[:, :, :3],
         jnp.zeros((B, Np, 10), jnp.float32)], axis=-1).reshape(B * Np, 16)

    sa1 = [_fold(br) for br in params["sa1"]]
    sa2 = [_fold(br) for br in params["sa2"]]
    sa3 = _fold(params["sa3"])

    nx1_cf, nx1, l1p = _sa_stage(
        pts_cf, table1, 6, 512, _SA1_RADII, _SA1_K, sa1, (64, 64, 32))
    table2 = jnp.concatenate(
        [l1p, nx1, jnp.zeros((B, 512, 13), jnp.float32)],
        axis=-1).reshape(B * 512, 176)
    _, nx2, l2p = _sa_stage(
        nx1_cf, table2, 163, 128, _SA2_RADII, _SA2_K, sa2, (32, 32, 16))
    t3 = jnp.concatenate([nx2, l2p], axis=-1)
    return _sa3(t3, sa3)
